# initial kernel scaffold (unmeasured)
import jax
import jax.numpy as jnp
from jax import lax
from jax.experimental import pallas as pl
from jax.experimental.pallas import tpu as pltpu


def kernel(
    x,
):
    def body(*refs):
        pass

    out_shape = jax.ShapeDtypeStruct(..., jnp.float32)
    return pl.pallas_call(body, out_shape=out_shape)(...)


# baseline (device time: 412625 ns/iter reference)
import jax
import jax.numpy as jnp
from jax import lax
from jax.experimental import pallas as pl
from jax.experimental.pallas import tpu as pltpu

_ADD_CHUNK = 1024


def kernel(x):
    m, n = x.shape
    half = m // 2
    n_chunks = half // _ADD_CHUNK

    def body(
        x_ref,
        out_ref,
        a_ref,
        b_ref,
        cp_sem_a,
        cp_sem_b,
        cp_sem_o,
        send_sem_x,
        recv_sem_x,
        send_sem_y,
        recv_sem_y,
    ):
        my_x = lax.axis_index("x")
        my_y = lax.axis_index("y")
        x_nbr = (1 - my_x, my_y)
        y_nbr = (my_x, 1 - my_y)

        barrier = pltpu.get_barrier_semaphore()
        for nbr in (x_nbr, y_nbr):
            pl.semaphore_signal(
                barrier, inc=1, device_id=nbr,
                device_id_type=pl.DeviceIdType.MESH,
            )
        pl.semaphore_wait(barrier, 2)

        my_off = my_y * half

        rdma_x = pltpu.make_async_remote_copy(
            src_ref=x_ref.at[pl.ds(my_off, half), :],
            dst_ref=out_ref.at[pl.ds(my_off, half), :],
            send_sem=send_sem_x,
            recv_sem=recv_sem_x,
            device_id=x_nbr,
            device_id_type=pl.DeviceIdType.MESH,
        )
        rdma_x.start()
        rdma_x.wait()

        for c in range(n_chunks):
            row = c * _ADD_CHUNK
            cp_a = pltpu.make_async_copy(
                x_ref.at[pl.ds(my_off + row, _ADD_CHUNK), :], a_ref, cp_sem_a
            )
            cp_b = pltpu.make_async_copy(
                out_ref.at[pl.ds(my_off + row, _ADD_CHUNK), :], b_ref, cp_sem_b
            )
            cp_a.start()
            cp_b.start()
            cp_a.wait()
            cp_b.wait()
            a_ref[...] = a_ref[...] + b_ref[...]
            cp_o = pltpu.make_async_copy(
                a_ref, out_ref.at[pl.ds(my_off + row, _ADD_CHUNK), :], cp_sem_o
            )
            cp_o.start()
            cp_o.wait()

        rdma_y = pltpu.make_async_remote_copy(
            src_ref=out_ref.at[pl.ds(my_off, half), :],
            dst_ref=out_ref.at[pl.ds(my_off, half), :],
            send_sem=send_sem_y,
            recv_sem=recv_sem_y,
            device_id=y_nbr,
            device_id_type=pl.DeviceIdType.MESH,
        )
        rdma_y.start()
        rdma_y.wait()

    return pl.pallas_call(
        body,
        out_shape=jax.ShapeDtypeStruct((m, n), jnp.float32),
        in_specs=[pl.BlockSpec(memory_space=pl.ANY)],
        out_specs=pl.BlockSpec(memory_space=pl.ANY),
        scratch_shapes=[
            pltpu.VMEM((_ADD_CHUNK, n), jnp.float32),
            pltpu.VMEM((_ADD_CHUNK, n), jnp.float32),
            pltpu.SemaphoreType.DMA,
            pltpu.SemaphoreType.DMA,
            pltpu.SemaphoreType.DMA,
            pltpu.SemaphoreType.DMA,
            pltpu.SemaphoreType.DMA,
            pltpu.SemaphoreType.DMA,
            pltpu.SemaphoreType.DMA,
        ],
        compiler_params=pltpu.CompilerParams(collective_id=0),
    )(x)


# device time: 224640 ns/iter; 1.8368x vs baseline; 1.8368x over previous
import jax
import jax.numpy as jnp
from jax import lax
from jax.experimental import pallas as pl
from jax.experimental.pallas import tpu as pltpu

_N_CHUNKS = 16


def kernel(x):
    m, n = x.shape
    half = m // 2
    ch = half // _N_CHUNKS

    def body(
        x_ref,
        out_ref,
        a_ref,
        b_ref,
        cp_sem_a,
        cp_sem_b,
        cp_sem_o,
        send_sems_x,
        recv_sems_x,
        send_sems_y,
        recv_sems_y,
    ):
        my_x = lax.axis_index("x")
        my_y = lax.axis_index("y")
        x_nbr = (1 - my_x, my_y)
        y_nbr = (my_x, 1 - my_y)

        barrier = pltpu.get_barrier_semaphore()
        for nbr in (x_nbr, y_nbr):
            pl.semaphore_signal(
                barrier, inc=1, device_id=nbr,
                device_id_type=pl.DeviceIdType.MESH,
            )
        pl.semaphore_wait(barrier, 2)

        my_off = my_y * half

        x_rdmas = []
        for k in range(_N_CHUNKS):
            row = my_off + k * ch
            rdma = pltpu.make_async_remote_copy(
                src_ref=x_ref.at[pl.ds(row, ch), :],
                dst_ref=out_ref.at[pl.ds(row, ch), :],
                send_sem=send_sems_x.at[k],
                recv_sem=recv_sems_x.at[k],
                device_id=x_nbr,
                device_id_type=pl.DeviceIdType.MESH,
            )
            rdma.start()
            x_rdmas.append(rdma)

        y_rdmas = []
        for k in range(_N_CHUNKS):
            row = my_off + k * ch
            x_rdmas[k].wait_recv()
            cp_a = pltpu.make_async_copy(
                x_ref.at[pl.ds(row, ch), :], a_ref, cp_sem_a
            )
            cp_b = pltpu.make_async_copy(
                out_ref.at[pl.ds(row, ch), :], b_ref, cp_sem_b
            )
            cp_a.start()
            cp_b.start()
            cp_a.wait()
            cp_b.wait()
            a_ref[...] = a_ref[...] + b_ref[...]
            cp_o = pltpu.make_async_copy(
                a_ref, out_ref.at[pl.ds(row, ch), :], cp_sem_o
            )
            cp_o.start()
            cp_o.wait()
            rdma = pltpu.make_async_remote_copy(
                src_ref=out_ref.at[pl.ds(row, ch), :],
                dst_ref=out_ref.at[pl.ds(row, ch), :],
                send_sem=send_sems_y.at[k],
                recv_sem=recv_sems_y.at[k],
                device_id=y_nbr,
                device_id_type=pl.DeviceIdType.MESH,
            )
            rdma.start()
            y_rdmas.append(rdma)

        for k in range(_N_CHUNKS):
            x_rdmas[k].wait_send()
            y_rdmas[k].wait_send()
        for k in range(_N_CHUNKS):
            y_rdmas[k].wait_recv()

    return pl.pallas_call(
        body,
        out_shape=jax.ShapeDtypeStruct((m, n), jnp.float32),
        in_specs=[pl.BlockSpec(memory_space=pl.ANY)],
        out_specs=pl.BlockSpec(memory_space=pl.ANY),
        scratch_shapes=[
            pltpu.VMEM((ch, n), jnp.float32),
            pltpu.VMEM((ch, n), jnp.float32),
            pltpu.SemaphoreType.DMA,
            pltpu.SemaphoreType.DMA,
            pltpu.SemaphoreType.DMA,
            pltpu.SemaphoreType.DMA((_N_CHUNKS,)),
            pltpu.SemaphoreType.DMA((_N_CHUNKS,)),
            pltpu.SemaphoreType.DMA((_N_CHUNKS,)),
            pltpu.SemaphoreType.DMA((_N_CHUNKS,)),
        ],
        compiler_params=pltpu.CompilerParams(collective_id=0),
    )(x)


# device time: 221717 ns/iter; 1.8610x vs baseline; 1.0132x over previous
import jax
import jax.numpy as jnp
from jax import lax
from jax.experimental import pallas as pl
from jax.experimental.pallas import tpu as pltpu

_N_CHUNKS = 16


def kernel(x):
    m, n = x.shape
    half = m // 2
    ch = half // _N_CHUNKS

    def body(
        x_ref,
        out_ref,
        recv_ref,
        a_ref,
        cp_sems_a,
        cp_sems_o,
        send_sems_x,
        recv_sems_x,
        send_sems_y,
        recv_sems_y,
    ):
        my_x = lax.axis_index("x")
        my_y = lax.axis_index("y")
        x_nbr = (1 - my_x, my_y)
        y_nbr = (my_x, 1 - my_y)

        barrier = pltpu.get_barrier_semaphore()
        for nbr in (x_nbr, y_nbr):
            pl.semaphore_signal(
                barrier, inc=1, device_id=nbr,
                device_id_type=pl.DeviceIdType.MESH,
            )
        pl.semaphore_wait(barrier, 2)

        my_off = my_y * half

        x_rdmas = []
        for k in range(_N_CHUNKS):
            rdma = pltpu.make_async_remote_copy(
                src_ref=x_ref.at[pl.ds(my_off + k * ch, ch), :],
                dst_ref=recv_ref.at[pl.ds(k * ch, ch), :],
                send_sem=send_sems_x.at[k],
                recv_sem=recv_sems_x.at[k],
                device_id=x_nbr,
                device_id_type=pl.DeviceIdType.MESH,
            )
            rdma.start()
            x_rdmas.append(rdma)

        def own_copy(k, slot):
            return pltpu.make_async_copy(
                x_ref.at[pl.ds(my_off + k * ch, ch), :],
                a_ref.at[slot],
                cp_sems_a.at[slot],
            )

        own_copy(0, 0).start()

        y_rdmas = []
        out_cps = []
        for k in range(_N_CHUNKS):
            slot = k % 2
            if k + 1 < _N_CHUNKS:
                own_copy(k + 1, 1 - slot).start()
            own_copy(k, slot).wait()
            x_rdmas[k].wait_recv()
            rows = pl.ds(k * ch, ch)
            recv_ref[rows, :] = recv_ref[rows, :] + a_ref[slot]
            rdma = pltpu.make_async_remote_copy(
                src_ref=recv_ref.at[rows, :],
                dst_ref=out_ref.at[pl.ds(my_off + k * ch, ch), :],
                send_sem=send_sems_y.at[k],
                recv_sem=recv_sems_y.at[k],
                device_id=y_nbr,
                device_id_type=pl.DeviceIdType.MESH,
            )
            rdma.start()
            y_rdmas.append(rdma)
            cp_o = pltpu.make_async_copy(
                recv_ref.at[rows, :],
                out_ref.at[pl.ds(my_off + k * ch, ch), :],
                cp_sems_o.at[k],
            )
            cp_o.start()
            out_cps.append(cp_o)

        for k in range(_N_CHUNKS):
            out_cps[k].wait()
            x_rdmas[k].wait_send()
            y_rdmas[k].wait_send()
        for k in range(_N_CHUNKS):
            y_rdmas[k].wait_recv()

    return pl.pallas_call(
        body,
        out_shape=jax.ShapeDtypeStruct((m, n), jnp.float32),
        in_specs=[pl.BlockSpec(memory_space=pl.ANY)],
        out_specs=pl.BlockSpec(memory_space=pl.ANY),
        scratch_shapes=[
            pltpu.VMEM((half, n), jnp.float32),
            pltpu.VMEM((2, ch, n), jnp.float32),
            pltpu.SemaphoreType.DMA((2,)),
            pltpu.SemaphoreType.DMA((_N_CHUNKS,)),
            pltpu.SemaphoreType.DMA((_N_CHUNKS,)),
            pltpu.SemaphoreType.DMA((_N_CHUNKS,)),
            pltpu.SemaphoreType.DMA((_N_CHUNKS,)),
            pltpu.SemaphoreType.DMA((_N_CHUNKS,)),
        ],
        compiler_params=pltpu.CompilerParams(collective_id=0),
    )(x)
